# baseline (speedup 1.0000x reference)
import functools
import math

import jax
import jax.numpy as jnp
from jax import lax
from jax.experimental import pallas as pl
from jax.experimental.pallas import tpu as pltpu
from jax.experimental.pallas import tpu_sc as plsc

_N = 2048
_V = 32000
_PAD = 0
_SMOOTH = 0.1
_CONF = 1.0 - _SMOOTH
_S = _SMOOTH / (_V - 2)
_K = (_V - 2) * _S * math.log(_S) + _CONF * math.log(_CONF)

_R = 256
_C = 6400

_NW = 32
_BPW = _N // _NW
_L = 16


def _loss_body(t_ref, x_ref, o_ref):
    i = pl.program_id(0)
    j = pl.program_id(1)

    @pl.when((i == 0) & (j == 0))
    def _init():
        o_ref[0, 0] = 0.0

    t = t_ref[...]
    valid = (t != _PAD)
    xb = x_ref[...]

    rows = jnp.sum(xb, axis=1, keepdims=True)
    rsum = jnp.sum(jnp.where(valid, rows, 0.0))

    @pl.when(j == 0)
    def _const():
        x0 = xb[:, 0:1]
        o_ref[0, 0] += jnp.sum(
            jnp.where(valid, jnp.float32(_K) + jnp.float32(_S) * x0, 0.0))

    o_ref[0, 0] += -jnp.float32(_S) * rsum


def _tc_loss(x, t2):
    nr = _N // _R
    nc = _V // _C
    out = pl.pallas_call(
        _loss_body,
        grid=(nr, nc),
        in_specs=[
            pl.BlockSpec((_R, 1), lambda i, j: (i, 0)),
            pl.BlockSpec((_R, _C), lambda i, j: (i, j)),
        ],
        out_specs=pl.BlockSpec((1, 1), lambda i, j: (0, 0),
                               memory_space=pltpu.SMEM),
        out_shape=jax.ShapeDtypeStruct((1, 1), jnp.float32),
        compiler_params=pltpu.CompilerParams(
            dimension_semantics=("arbitrary", "arbitrary")),
    )(t2, x)
    return out[0, 0]


def _sc_gather_body(x_hbm, t_hbm, out_hbm, tloc_v, idx_v, vals_v, acc_v, sem):
    wid = lax.axis_index("s") * 2 + lax.axis_index("c")
    base = wid * _BPW

    pltpu.sync_copy(t_hbm.at[pl.ds(base, _BPW)], tloc_v)

    lanes = lax.iota(jnp.int32, _L)
    for k in range(_BPW // _L):
        tv = tloc_v[pl.ds(k * _L, _L)]
        rows = (base + k * _L) + lanes
        idx_v[pl.ds(k * _L, _L)] = rows * jnp.int32(_V) + tv

    pltpu.async_copy(x_hbm.at[idx_v], vals_v, sem).wait()

    acc = jnp.zeros((_L,), jnp.float32)
    for k in range(_BPW // _L):
        tv = tloc_v[pl.ds(k * _L, _L)]
        vv = vals_v[pl.ds(k * _L, _L)]
        acc = acc + jnp.where(tv != _PAD, vv, 0.0)
    acc_v[...] = acc * jnp.float32(_S - _CONF)
    pltpu.sync_copy(acc_v, out_hbm.at[wid])


def _sc_gather(xf, t32):
    mesh = plsc.VectorSubcoreMesh(core_axis_name="c", subcore_axis_name="s")
    fn = functools.partial(
        pl.kernel,
        mesh=mesh,
        out_type=jax.ShapeDtypeStruct((_NW, _L), jnp.float32),
        scratch_types=[
            pltpu.VMEM((_BPW,), jnp.int32),
            pltpu.VMEM((_BPW,), jnp.int32),
            pltpu.VMEM((_BPW,), jnp.float32),
            pltpu.VMEM((_L,), jnp.float32),
            pltpu.SemaphoreType.DMA,
        ],
    )(_sc_gather_body)
    return fn(xf, t32)


def kernel(x, target):
    t32 = target.astype(jnp.int32)
    t2 = t32.reshape(_N, 1)
    xf = x.reshape(_N * _V)
    sc_part = _sc_gather(xf, t32)
    tc_part = _tc_loss(x, t2)
    return tc_part + jnp.sum(sc_part)


# speedup vs baseline: 2.5668x; 2.5668x over previous
import functools
import math

import jax
import jax.numpy as jnp
from jax.experimental import pallas as pl
from jax.experimental.pallas import tpu as pltpu

_N = 2048
_V = 32000
_PAD = 0
_SMOOTH = 0.1
_CONF = 1.0 - _SMOOTH
_S = _SMOOTH / (_V - 2)
_K = (_V - 2) * _S * math.log(_S) + _CONF * math.log(_CONF)

_R = 256
_C = 16000


def _loss_body(t_ref, x_ref, o_ref):
    i = pl.program_id(0)
    j = pl.program_id(1)

    @pl.when((i == 0) & (j == 0))
    def _init():
        o_ref[0, 0] = 0.0

    t = t_ref[...]
    valid = (t != _PAD)
    xb = x_ref[...]

    tloc = jnp.where(valid, t - j * _C, -1)
    iota = jax.lax.broadcasted_iota(jnp.int32, (_R, _C), 1)
    gath = jnp.sum(jnp.where(iota == tloc, xb, 0.0))

    rows = jnp.sum(xb, axis=1, keepdims=True)
    rsum = jnp.sum(jnp.where(valid, rows, 0.0))

    partial = jnp.float32(_S - _CONF) * gath - jnp.float32(_S) * rsum

    @pl.when(j == 0)
    def _const():
        x0 = xb[:, 0:1]
        o_ref[0, 0] += jnp.sum(
            jnp.where(valid, jnp.float32(_K) + jnp.float32(_S) * x0, 0.0))

    o_ref[0, 0] += partial


def kernel(x, target):
    nr = _N // _R
    nc = _V // _C
    t2 = target.astype(jnp.int32).reshape(_N, 1)
    out = pl.pallas_call(
        _loss_body,
        grid=(nr, nc),
        in_specs=[
            pl.BlockSpec((_R, 1), lambda i, j: (i, 0)),
            pl.BlockSpec((_R, _C), lambda i, j: (i, j)),
        ],
        out_specs=pl.BlockSpec((1, 1), lambda i, j: (0, 0),
                               memory_space=pltpu.SMEM),
        out_shape=jax.ShapeDtypeStruct((1, 1), jnp.float32),
        compiler_params=pltpu.CompilerParams(
            dimension_semantics=("arbitrary", "arbitrary")),
    )(t2, x)
    return out[0, 0]
